# Initial kernel scaffold; baseline (speedup 1.0000x reference)
import jax
import jax.numpy as jnp
from jax.experimental import pallas as pl


def kernel(text_seqs, chars, text_table, char_table, W, b):
    raise NotImplementedError("write your pallas kernel here")


# speedup vs baseline: 2.1450x; 2.1450x over previous
import functools
import jax
import jax.numpy as jnp
from jax import lax
from jax.experimental import pallas as pl
from jax.experimental.pallas import tpu as pltpu
from jax.experimental.pallas import tpu_sc as plsc

D = 64
TEXT_VOCAB = 100000
CHAR_VOCAB = 1000
B, L = 4096, 200
NW = 32
BT = B // NW
NBC = BT // 16


def _mm_body(x_ref, w_ref, o_ref):
    o_ref[:] = jnp.dot(x_ref[:], w_ref[:], preferred_element_type=jnp.float32) * 8.0


def _mm_bias_body(x_ref, w_ref, b_ref, o_ref):
    y = (
        jnp.dot(x_ref[:], w_ref[:], preferred_element_type=jnp.float32) * 8.0
        + b_ref[:]
    )
    o_ref[:] = jnp.concatenate([y, y], axis=1)


def _fold_text_pairs(x2, W2):
    blk = 2000
    n = TEXT_VOCAB // 2
    return pl.pallas_call(
        _mm_body,
        grid=(n // blk,),
        in_specs=[
            pl.BlockSpec((blk, 128), lambda i: (i, 0)),
            pl.BlockSpec((128, 128), lambda i: (0, 0)),
        ],
        out_specs=pl.BlockSpec((blk, 128), lambda i: (i, 0)),
        out_shape=jax.ShapeDtypeStruct((n, 128), jnp.float32),
    )(x2, W2)


def _fold_char_table(char_table, Wc, b2):
    return pl.pallas_call(
        _mm_bias_body,
        out_shape=jax.ShapeDtypeStruct((CHAR_VOCAB, 128), jnp.float32),
    )(char_table, Wc, b2)


def _sc_body(t2_hbm, c2_hbm, idx_hbm, chars_hbm, out_hbm,
             idx_v, kidxA, kidxB, parA, parB, rowsA, rowsB, outA, outB,
             ct_v, cidx_v, semA, semB, semWA, semWB):
    wid = lax.axis_index("s") * 2 + lax.axis_index("c")
    iota = lax.iota(jnp.int32, 16)

    pltpu.sync_copy(idx_hbm.at[pl.ds(wid * L, L)], idx_v)
    pltpu.sync_copy(chars_hbm.at[pl.ds(wid * BT, BT)], cidx_v)

    pltpu.async_copy(c2_hbm.at[cidx_v], rowsA, semA).wait()
    for bc in range(NBC):
        slot16 = iota + bc * 16

        def ct_body(d, ccol, _bc=bc, _slot=slot16):
            ct_v[d, pl.ds(_bc * 16, 16)] = plsc.load_gather(rowsA, [_slot, ccol])
            return ccol + 1

        lax.fori_loop(0, D, ct_body, jnp.zeros((16,), jnp.int32))

    def prep(l, kidx_v, par_v):
        for c in range(NBC):
            chunk = idx_v[l, pl.ds(c * 16, 16)]
            kidx_v[pl.ds(c * 16, 16)] = lax.shift_right_logical(chunk, 1)
            par_v[pl.ds(c * 16, 16)] = (chunk & 1) * D

    def transpose_add(rows_v, par_v, out_v):
        for bc in range(NBC):
            slot16 = iota + bc * 16
            p0 = par_v[pl.ds(bc * 16, 16)]

            def dbody(d, pcol, _bc=bc, _slot=slot16, _rows=rows_v, _out=out_v):
                val = plsc.load_gather(_rows, [_slot, pcol])
                _out[d, pl.ds(_bc * 16, 16)] = val + ct_v[d, pl.ds(_bc * 16, 16)]
                return pcol + 1

            lax.fori_loop(0, D, dbody, p0)

    out_col = pl.ds(wid * BT, BT)

    prep(0, kidxA, parA)
    pltpu.async_copy(t2_hbm.at[kidxA], rowsA, semA)
    prep(1, kidxB, parB)
    pltpu.async_copy(t2_hbm.at[kidxB], rowsB, semB)

    def body(i, carry):
        lA = 2 * i
        lB = 2 * i + 1
        pltpu.make_async_copy(t2_hbm.at[kidxA], rowsA, semA).wait()

        @pl.when(i > 0)
        def _():
            pltpu.make_async_copy(outA, out_hbm.at[0, :, out_col], semWA).wait()

        transpose_add(rowsA, parA, outA)
        pltpu.async_copy(outA, out_hbm.at[lA, :, out_col], semWA)
        prep(jnp.minimum(lA + 2, L - 1), kidxA, parA)
        pltpu.async_copy(t2_hbm.at[kidxA], rowsA, semA)
        pltpu.make_async_copy(t2_hbm.at[kidxB], rowsB, semB).wait()

        @pl.when(i > 0)
        def _():
            pltpu.make_async_copy(outB, out_hbm.at[0, :, out_col], semWB).wait()

        transpose_add(rowsB, parB, outB)
        pltpu.async_copy(outB, out_hbm.at[lB, :, out_col], semWB)
        prep(jnp.minimum(lB + 2, L - 1), kidxB, parB)
        pltpu.async_copy(t2_hbm.at[kidxB], rowsB, semB)
        return carry

    lax.fori_loop(0, L // 2, body, 0)

    pltpu.make_async_copy(t2_hbm.at[kidxA], rowsA, semA).wait()
    pltpu.make_async_copy(t2_hbm.at[kidxB], rowsB, semB).wait()
    pltpu.make_async_copy(outA, out_hbm.at[0, :, out_col], semWA).wait()
    pltpu.make_async_copy(outB, out_hbm.at[0, :, out_col], semWB).wait()


def _sc_gather_transpose(T2, C2, IDX, chars):
    mesh = plsc.VectorSubcoreMesh(core_axis_name="c", subcore_axis_name="s")
    f = functools.partial(
        pl.kernel,
        mesh=mesh,
        compiler_params=pltpu.CompilerParams(
            use_tc_tiling_on_sc=True, needs_layout_passes=False
        ),
        out_type=jax.ShapeDtypeStruct((L, D, B), jnp.float32),
        scratch_types=[
            pltpu.VMEM((L, BT), jnp.int32),
            pltpu.VMEM((BT,), jnp.int32),
            pltpu.VMEM((BT,), jnp.int32),
            pltpu.VMEM((BT,), jnp.int32),
            pltpu.VMEM((BT,), jnp.int32),
            pltpu.VMEM((BT, 128), jnp.float32),
            pltpu.VMEM((BT, 128), jnp.float32),
            pltpu.VMEM((D, BT), jnp.float32),
            pltpu.VMEM((D, BT), jnp.float32),
            pltpu.VMEM((D, BT), jnp.float32),
            pltpu.VMEM((BT,), jnp.int32),
            pltpu.SemaphoreType.DMA,
            pltpu.SemaphoreType.DMA,
            pltpu.SemaphoreType.DMA,
            pltpu.SemaphoreType.DMA,
        ],
    )(_sc_body)
    return f(T2, C2, IDX, chars)


def kernel(text_seqs, chars, text_table, char_table, W, b):
    Wt = W[:D]
    Wc = W[D:]
    W2 = jnp.zeros((128, 128), jnp.float32)
    W2 = W2.at[:D, :D].set(Wt).at[D:, D:].set(Wt)
    x2 = text_table.reshape(TEXT_VOCAB // 2, 128)
    T2 = _fold_text_pairs(x2, W2)
    C2 = _fold_char_table(char_table, Wc, b.reshape(1, D))
    IDX = (
        text_seqs.astype(jnp.int32)
        .reshape(NW, BT, L)
        .transpose(0, 2, 1)
        .reshape(NW * L, BT)
    )
    out_t = _sc_gather_transpose(T2, C2, IDX, chars.astype(jnp.int32))
    return out_t.transpose(2, 0, 1)
